# Initial kernel scaffold; baseline (speedup 1.0000x reference)
import jax
import jax.numpy as jnp
from jax.experimental import pallas as pl


def kernel(edges, weights, vertex_features, W1, b1, W2, b2, W3, b3):
    raise NotImplementedError("write your pallas kernel here")


# speedup vs baseline: 29.5021x; 29.5021x over previous
import functools

import jax
import jax.numpy as jnp
from jax import lax
from jax.experimental import pallas as pl
from jax.experimental.pallas import tpu as pltpu
from jax.experimental.pallas import tpu_sc as plsc

N_NODES = 50000
HID = 16
NC = 2
NS = 16
NW = NC * NS

ROW = 128
STEPS = 32
SUPER = ROW * STEPS

N_PAD = 50176
ZROWS = 224


def _edge_pass(src2d, dst2d, w, h, n_super):
  n = h.shape[0]
  per_sub = n // NS

  mesh = plsc.VectorSubcoreMesh(core_axis_name="c", subcore_axis_name="s")

  @functools.partial(
      pl.kernel,
      mesh=mesh,
      out_type=jax.ShapeDtypeStruct((NC, n, HID), jnp.float32),
      compiler_params=pltpu.CompilerParams(use_tc_tiling_on_sc=False),
      scratch_types=[
          pltpu.VMEM_SHARED((n, HID), jnp.float32),
          pltpu.VMEM((STEPS, ROW), jnp.int32),
          pltpu.VMEM((STEPS, ROW), jnp.int32),
          pltpu.VMEM((SUPER,), jnp.float32),
          pltpu.VMEM((ROW, HID), jnp.float32),
          pltpu.VMEM((ZROWS, HID), jnp.float32),
          pltpu.SemaphoreType.DMA,
      ],
  )
  def kfn(src_hbm, dst_hbm, w_hbm, h_hbm, out_hbm,
          acc_sh, src_v, dst_v, w_v, rows_v, zero_v, sem):
    c = lax.axis_index("c")
    s = lax.axis_index("s")
    wid = c * NS + s

    zbase = s * per_sub
    def zfill(i, carry):
      zero_v[i, :] = jnp.zeros((HID,), jnp.float32)
      return carry
    lax.fori_loop(0, ZROWS, zfill, 0)
    def zcopy(i, carry):
      pltpu.sync_copy(zero_v, acc_sh.at[pl.ds(zbase + i * ZROWS, ZROWS)])
      return carry
    lax.fori_loop(0, per_sub // ZROWS, zcopy, 0)
    plsc.subcore_barrier()

    def superstep(t, carry):
      rbase = (wid * n_super + t) * STEPS
      ebase = (wid * n_super + t) * SUPER
      pltpu.sync_copy(src_hbm.at[pl.ds(rbase, STEPS)], src_v)
      pltpu.sync_copy(dst_hbm.at[pl.ds(rbase, STEPS)], dst_v)
      pltpu.sync_copy(w_hbm.at[pl.ds(ebase, SUPER)], w_v)

      def step(j, carry2):
        pltpu.async_copy(h_hbm.at[src_v.at[j]], rows_v, sem).wait()

        def grp(g, carry3):
          w16 = w_v[pl.ds(j * ROW + g * 16, 16)]
          for i in range(16):
            e = g * 16 + i
            rows_v[e, :] = rows_v[e, :] * w16[i]
          return carry3
        lax.fori_loop(0, ROW // 16, grp, 0)

        pltpu.sync_copy(rows_v, acc_sh.at[dst_v.at[j]], add=True)
        return carry2
      lax.fori_loop(0, STEPS, step, 0)
      return carry
    lax.fori_loop(0, n_super, superstep, 0)
    plsc.subcore_barrier()

    pltpu.sync_copy(acc_sh.at[pl.ds(s * per_sub, per_sub)],
                    out_hbm.at[c].at[pl.ds(s * per_sub, per_sub)])

  return kfn(src2d, dst2d, w, h)


def _tc_prep(deg_agg, x0, w1):
  n = x0.shape[0]
  blk = 1568
  grid = (n // blk,)

  def body(dega_ref, x0_ref, w1_ref, dinv_ref, h1p_ref):
    deg = dega_ref[0, :, 0:1] + dega_ref[1, :, 0:1] + 1.0
    dinv = lax.rsqrt(deg)
    dinv_ref[:, :] = dinv
    h1p_ref[:, :] = jnp.dot(x0_ref[:, :], w1_ref[:, :],
                            preferred_element_type=jnp.float32) * dinv

  return pl.pallas_call(
      body,
      grid=grid,
      in_specs=[
          pl.BlockSpec((NC, blk, HID), lambda i: (0, i, 0)),
          pl.BlockSpec((blk, 4), lambda i: (i, 0)),
          pl.BlockSpec((4, HID), lambda i: (0, 0)),
      ],
      out_specs=[
          pl.BlockSpec((blk, 1), lambda i: (i, 0)),
          pl.BlockSpec((blk, HID), lambda i: (i, 0)),
      ],
      out_shape=[
          jax.ShapeDtypeStruct((n, 1), jnp.float32),
          jax.ShapeDtypeStruct((n, HID), jnp.float32),
      ],
  )(deg_agg, x0, w1)


def _tc_layer(agg, hp, dinv, b, wnext):
  n = hp.shape[0]
  blk = 1568
  grid = (n // blk,)
  fo = wnext.shape[1]

  def body(agg_ref, hp_ref, dinv_ref, b_ref, wn_ref, out_ref):
    a = agg_ref[0, :, :] + agg_ref[1, :, :] + hp_ref[:, :]
    x = jnp.maximum(dinv_ref[:, :] * a + b_ref[:, :], 0.0)
    out_ref[:, :] = jnp.dot(x, wn_ref[:, :],
                            preferred_element_type=jnp.float32) * dinv_ref[:, :]

  return pl.pallas_call(
      body,
      grid=grid,
      in_specs=[
          pl.BlockSpec((NC, blk, HID), lambda i: (0, i, 0)),
          pl.BlockSpec((blk, HID), lambda i: (i, 0)),
          pl.BlockSpec((blk, 1), lambda i: (i, 0)),
          pl.BlockSpec((1, HID), lambda i: (0, 0)),
          pl.BlockSpec((HID, fo), lambda i: (0, 0)),
      ],
      out_specs=pl.BlockSpec((blk, fo), lambda i: (i, 0)),
      out_shape=jax.ShapeDtypeStruct((n, fo), jnp.float32),
  )(agg, hp, dinv, b, wnext)


def _tc_final(agg, hp, dinv, b3, n_real):
  n = hp.shape[0]
  blk = 1568
  grid = (n // blk,)

  def body(agg_ref, hp_ref, dinv_ref, b3_ref, out_ref):
    a = agg_ref[0, :, 0:1] + agg_ref[1, :, 0:1] + hp_ref[:, 0:1]
    x = jnp.maximum(dinv_ref[:, :] * a + b3_ref[:, :], 0.0)
    rows = (lax.broadcasted_iota(jnp.int32, (blk, 1), 0)
            + pl.program_id(0) * blk)
    x = jnp.where(rows < n_real, x, 0.0)
    part = jnp.sum(x) * (1.0 / n_real)

    @pl.when(pl.program_id(0) == 0)
    def _():
      out_ref[:, :] = jnp.zeros((1, 1), jnp.float32)

    out_ref[:, :] = out_ref[:, :] + part

  return pl.pallas_call(
      body,
      grid=grid,
      in_specs=[
          pl.BlockSpec((NC, blk, HID), lambda i: (0, i, 0)),
          pl.BlockSpec((blk, HID), lambda i: (i, 0)),
          pl.BlockSpec((blk, 1), lambda i: (i, 0)),
          pl.BlockSpec((1, 1), lambda i: (0, 0)),
      ],
      out_specs=pl.BlockSpec((1, 1), lambda i: (0, 0)),
      out_shape=jax.ShapeDtypeStruct((1, 1), jnp.float32),
  )(agg, hp, dinv, b3)


def kernel(edges, weights, vertex_features, W1, b1, W2, b2, W3, b3):
  src = edges[0]
  dst = edges[1]
  n = vertex_features.shape[0]
  e = src.shape[0]

  n_super = -(-e // (NW * SUPER))
  e_pad = NW * n_super * SUPER
  pad = e_pad - e
  src_p = jnp.concatenate([src, jnp.zeros((pad,), jnp.int32)])
  dst_p = jnp.concatenate([dst, jnp.zeros((pad,), jnp.int32)])
  w_p = jnp.concatenate([weights, jnp.zeros((pad,), jnp.float32)])
  src2d = src_p.reshape(e_pad // ROW, ROW)
  dst2d = dst_p.reshape(e_pad // ROW, ROW)

  x0_p = jnp.pad(vertex_features, ((0, N_PAD - n), (0, 0)))

  ones_h = jnp.ones((N_PAD, HID), jnp.float32)
  deg_agg = _edge_pass(src2d, dst2d, w_p, ones_h, n_super)

  dinv, h1p = _tc_prep(deg_agg, x0_p, W1)

  agg1 = _edge_pass(src2d, dst2d, w_p, h1p, n_super)
  h2p = _tc_layer(agg1, h1p, dinv, b1.reshape(1, HID), W2)

  agg2 = _edge_pass(src2d, dst2d, w_p, h2p, n_super)
  w3pad = jnp.pad(W3, ((0, 0), (0, HID - W3.shape[1])))
  h3p = _tc_layer(agg2, h2p, dinv, b2.reshape(1, HID), w3pad)

  agg3 = _edge_pass(src2d, dst2d, w_p, h3p, n_super)
  q = _tc_final(agg3, h3p, dinv, b3.reshape(1, 1), n)
  return q


# speedup vs baseline: 50.2502x; 1.7033x over previous
import functools

import jax
import jax.numpy as jnp
from jax import lax
from jax.experimental import pallas as pl
from jax.experimental.pallas import tpu as pltpu
from jax.experimental.pallas import tpu_sc as plsc

HID = 16
NC = 2
NS = 16
NW = NC * NS

ROW = 128
STEPS = 32
SUPER = ROW * STEPS

N_PAD = 50176
ZROWS = 224
ZROWS1 = 784


def _sc_pass(variant, src2d, dst2d, w2d, h, n_super):
  wide = variant == "wide"
  gather = variant != "deg"
  n = N_PAD
  per_sub = n // NS

  mesh = plsc.VectorSubcoreMesh(core_axis_name="c", subcore_axis_name="s")

  if wide:
    acc_t = pltpu.VMEM_SHARED((n, HID), jnp.float32)
    out_t = jax.ShapeDtypeStruct((NC, n, HID), jnp.float32)
    buf_t = pltpu.VMEM((ROW, HID), jnp.float32)
    zero_t = pltpu.VMEM((ZROWS, HID), jnp.float32)
  else:
    acc_t = pltpu.VMEM_SHARED((n,), jnp.float32)
    out_t = jax.ShapeDtypeStruct((NC, n), jnp.float32)
    buf_t = pltpu.VMEM((ROW,), jnp.float32)
    zero_t = pltpu.VMEM((ZROWS1,), jnp.float32)

  scratch = [
      acc_t,
      pltpu.VMEM((STEPS, ROW), jnp.int32),
      pltpu.VMEM((STEPS, ROW), jnp.float32),
      zero_t,
      buf_t, buf_t,
      pltpu.SemaphoreType.DMA,
      pltpu.SemaphoreType.DMA,
  ]
  if gather:
    scratch.insert(1, pltpu.VMEM((STEPS, ROW), jnp.int32))

  @functools.partial(
      pl.kernel,
      mesh=mesh,
      out_type=out_t,
      compiler_params=pltpu.CompilerParams(use_tc_tiling_on_sc=False),
      scratch_types=scratch,
  )
  def kfn(*refs):
    if gather:
      (src_hbm, dst_hbm, w_hbm, h_hbm, out_hbm,
       acc_sh, src_v, dst_v, w_v, zero_v, buf_a, buf_b, gsem, ssem) = refs
    else:
      (dst_hbm, w_hbm, out_hbm,
       acc_sh, dst_v, w_v, zero_v, buf_a, buf_b, gsem, ssem) = refs
    c = lax.axis_index("c")
    s = lax.axis_index("s")
    wid = c * NS + s
    bufs = (buf_a, buf_b)

    zbase = s * per_sub
    zn = ZROWS if wide else ZROWS1
    def zfill(i, carry):
      if wide:
        zero_v[i, :] = jnp.zeros((HID,), jnp.float32)
      else:
        zero_v[pl.ds(i * 16, 16)] = jnp.zeros((16,), jnp.float32)
      return carry
    lax.fori_loop(0, zn if wide else zn // 16, zfill, 0)
    def zcopy(i, carry):
      pltpu.sync_copy(zero_v, acc_sh.at[pl.ds(zbase + i * zn, zn)])
      return carry
    lax.fori_loop(0, per_sub // zn, zcopy, 0)
    plsc.subcore_barrier()

    def scale(buf, j):
      def grp(g, carry):
        w16 = w_v[j, pl.ds(g * 16, 16)]
        if wide:
          for i in range(16):
            e = g * 16 + i
            buf[e, :] = buf[e, :] * w16[i]
        else:
          buf[pl.ds(g * 16, 16)] = buf[pl.ds(g * 16, 16)] * w16
        return carry
      lax.fori_loop(0, ROW // 16, grp, 0)

    def superstep(t, carry):
      rbase = (wid * n_super + t) * STEPS
      if gather:
        pltpu.sync_copy(src_hbm.at[pl.ds(rbase, STEPS)], src_v)
      pltpu.sync_copy(dst_hbm.at[pl.ds(rbase, STEPS)], dst_v)
      pltpu.sync_copy(w_hbm.at[pl.ds(rbase, STEPS)], w_v)

      if gather:
        gh = {}
        sh = {}
        gh[0] = pltpu.async_copy(h_hbm.at[src_v.at[0]], bufs[0], gsem)
        for j in range(STEPS):
          if j + 1 < STEPS:
            if j >= 1:
              sh[j - 1].wait()
            gh[j + 1] = pltpu.async_copy(
                h_hbm.at[src_v.at[j + 1]], bufs[(j + 1) % 2], gsem)
          gh[j].wait()
          scale(bufs[j % 2], j)
          sh[j] = pltpu.async_copy(
              bufs[j % 2], acc_sh.at[dst_v.at[j]], ssem, add=True)
        sh[STEPS - 2].wait()
        sh[STEPS - 1].wait()
      else:
        sh = {}
        for j in range(STEPS):
          if j >= 2:
            sh[j - 2].wait()
          sh[j] = pltpu.async_copy(
              w_v.at[j], acc_sh.at[dst_v.at[j]], ssem, add=True)
        sh[STEPS - 2].wait()
        sh[STEPS - 1].wait()
      return carry
    lax.fori_loop(0, n_super, superstep, 0)
    plsc.subcore_barrier()

    pltpu.sync_copy(acc_sh.at[pl.ds(zbase, per_sub)],
                    out_hbm.at[c].at[pl.ds(zbase, per_sub)])

  if gather:
    return kfn(src2d, dst2d, w2d, h)
  return kfn(dst2d, w2d)


def _tc_prep(deg_agg, x0, w1):
  n = x0.shape[0]
  blk = 1568
  grid = (n // blk,)

  def body(dega_ref, x0_ref, w1_ref, dinv_ref, h1p_ref):
    deg = dega_ref[0, :, :] + dega_ref[1, :, :] + 1.0
    dinv = lax.rsqrt(deg)
    dinv_ref[:, :] = dinv
    h1p_ref[:, :] = jnp.dot(x0_ref[:, :], w1_ref[:, :],
                            preferred_element_type=jnp.float32) * dinv

  return pl.pallas_call(
      body,
      grid=grid,
      in_specs=[
          pl.BlockSpec((NC, blk, 1), lambda i: (0, i, 0)),
          pl.BlockSpec((blk, 4), lambda i: (i, 0)),
          pl.BlockSpec((4, HID), lambda i: (0, 0)),
      ],
      out_specs=[
          pl.BlockSpec((blk, 1), lambda i: (i, 0)),
          pl.BlockSpec((blk, HID), lambda i: (i, 0)),
      ],
      out_shape=[
          jax.ShapeDtypeStruct((n, 1), jnp.float32),
          jax.ShapeDtypeStruct((n, HID), jnp.float32),
      ],
  )(deg_agg, x0, w1)


def _tc_layer(agg, hp, dinv, b, wnext):
  n = hp.shape[0]
  blk = 1568
  grid = (n // blk,)
  fo = wnext.shape[1]

  def body(agg_ref, hp_ref, dinv_ref, b_ref, wn_ref, out_ref):
    a = agg_ref[0, :, :] + agg_ref[1, :, :] + hp_ref[:, :]
    x = jnp.maximum(dinv_ref[:, :] * a + b_ref[:, :], 0.0)
    out_ref[:, :] = jnp.dot(x, wn_ref[:, :],
                            preferred_element_type=jnp.float32) * dinv_ref[:, :]

  return pl.pallas_call(
      body,
      grid=grid,
      in_specs=[
          pl.BlockSpec((NC, blk, HID), lambda i: (0, i, 0)),
          pl.BlockSpec((blk, HID), lambda i: (i, 0)),
          pl.BlockSpec((blk, 1), lambda i: (i, 0)),
          pl.BlockSpec((1, HID), lambda i: (0, 0)),
          pl.BlockSpec((HID, fo), lambda i: (0, 0)),
      ],
      out_specs=pl.BlockSpec((blk, fo), lambda i: (i, 0)),
      out_shape=jax.ShapeDtypeStruct((n, fo), jnp.float32),
  )(agg, hp, dinv, b, wnext)


def _tc_final(agg, hp, dinv, b3, n_real):
  n = hp.shape[0]
  blk = 1568
  grid = (n // blk,)

  def body(agg_ref, hp_ref, dinv_ref, b3_ref, out_ref):
    a = agg_ref[0, :, :] + agg_ref[1, :, :] + hp_ref[:, :]
    x = jnp.maximum(dinv_ref[:, :] * a + b3_ref[:, :], 0.0)
    rows = (lax.broadcasted_iota(jnp.int32, (blk, 1), 0)
            + pl.program_id(0) * blk)
    x = jnp.where(rows < n_real, x, 0.0)
    part = jnp.sum(x) * (1.0 / n_real)

    @pl.when(pl.program_id(0) == 0)
    def _():
      out_ref[:, :] = jnp.zeros((1, 1), jnp.float32)

    out_ref[:, :] = out_ref[:, :] + part

  return pl.pallas_call(
      body,
      grid=grid,
      in_specs=[
          pl.BlockSpec((NC, blk, 1), lambda i: (0, i, 0)),
          pl.BlockSpec((blk, 1), lambda i: (i, 0)),
          pl.BlockSpec((blk, 1), lambda i: (i, 0)),
          pl.BlockSpec((1, 1), lambda i: (0, 0)),
      ],
      out_specs=pl.BlockSpec((1, 1), lambda i: (0, 0)),
      out_shape=jax.ShapeDtypeStruct((1, 1), jnp.float32),
  )(agg, hp, dinv, b3)


def kernel(edges, weights, vertex_features, W1, b1, W2, b2, W3, b3):
  src = edges[0]
  dst = edges[1]
  n = vertex_features.shape[0]
  e = src.shape[0]

  n_super = -(-e // (NW * SUPER))
  e_pad = NW * n_super * SUPER
  pad = e_pad - e
  src_p = jnp.concatenate([src, jnp.zeros((pad,), jnp.int32)])
  dst_p = jnp.concatenate([dst, jnp.zeros((pad,), jnp.int32)])
  w_p = jnp.concatenate([weights, jnp.zeros((pad,), jnp.float32)])
  src2d = src_p.reshape(e_pad // ROW, ROW)
  dst2d = dst_p.reshape(e_pad // ROW, ROW)
  w2d = w_p.reshape(e_pad // ROW, ROW)

  x0_p = jnp.pad(vertex_features, ((0, N_PAD - n), (0, 0)))

  deg_agg = _sc_pass("deg", None, dst2d, w2d, None, n_super)

  dinv, h1p = _tc_prep(deg_agg.reshape(NC, N_PAD, 1), x0_p, W1)

  agg1 = _sc_pass("wide", src2d, dst2d, w2d, h1p, n_super)
  h2p = _tc_layer(agg1, h1p, dinv, b1.reshape(1, HID), W2)

  agg2 = _sc_pass("wide", src2d, dst2d, w2d, h2p, n_super)
  h3p = _tc_layer(agg2, h2p, dinv, b2.reshape(1, HID), W3)

  agg3 = _sc_pass("elem", src2d, dst2d, w2d, h3p.reshape(N_PAD), n_super)
  q = _tc_final(agg3.reshape(NC, N_PAD, 1), h3p, dinv, b3.reshape(1, 1), n)
  return q


# speedup vs baseline: 55.3575x; 1.1016x over previous
import functools

import jax
import jax.numpy as jnp
from jax import lax
from jax.experimental import pallas as pl
from jax.experimental.pallas import tpu as pltpu
from jax.experimental.pallas import tpu_sc as plsc

HID = 16
NC = 2
NS = 16
NW = NC * NS

ROW = 128
SUB = 8
NSTR = 4
STEPS = SUB * NSTR
SUPER = ROW * STEPS
SROW = SUB * ROW

N_PAD = 50176
ZROWS = 224
ZROWS1 = 784


def _sc_pass(variant, src2d, dst2d, w2d, h, n_super):
  wide = variant == "wide"
  gather = variant != "deg"
  n = N_PAD
  per_sub = n // NS

  mesh = plsc.VectorSubcoreMesh(core_axis_name="c", subcore_axis_name="s")

  if wide:
    acc_t = pltpu.VMEM_SHARED((n, HID), jnp.float32)
    out_t = jax.ShapeDtypeStruct((NC, n, HID), jnp.float32)
    buf_t = pltpu.VMEM((SROW, HID), jnp.float32)
    zero_t = pltpu.VMEM((ZROWS, HID), jnp.float32)
  else:
    acc_t = pltpu.VMEM_SHARED((n,), jnp.float32)
    out_t = jax.ShapeDtypeStruct((NC, n), jnp.float32)
    buf_t = pltpu.VMEM((SROW,), jnp.float32)
    zero_t = pltpu.VMEM((ZROWS1,), jnp.float32)

  scratch = [
      acc_t,
      pltpu.VMEM((NSTR, SROW), jnp.int32),
      pltpu.VMEM((NSTR, SROW), jnp.float32),
      zero_t,
      buf_t, buf_t,
      pltpu.SemaphoreType.DMA,
      pltpu.SemaphoreType.DMA,
  ]
  if gather:
    scratch.insert(1, pltpu.VMEM((NSTR, SROW), jnp.int32))

  @functools.partial(
      pl.kernel,
      mesh=mesh,
      out_type=out_t,
      compiler_params=pltpu.CompilerParams(use_tc_tiling_on_sc=False),
      scratch_types=scratch,
  )
  def kfn(*refs):
    if gather:
      (src_hbm, dst_hbm, w_hbm, h_hbm, out_hbm,
       acc_sh, src_v, dst_v, w_v, zero_v, buf_a, buf_b, gsem, ssem) = refs
    else:
      (dst_hbm, w_hbm, out_hbm,
       acc_sh, dst_v, w_v, zero_v, buf_a, buf_b, gsem, ssem) = refs
    c = lax.axis_index("c")
    s = lax.axis_index("s")
    wid = c * NS + s
    bufs = (buf_a, buf_b)

    zbase = s * per_sub
    zn = ZROWS if wide else ZROWS1
    def zfill(i, carry):
      if wide:
        zero_v[i, :] = jnp.zeros((HID,), jnp.float32)
      else:
        zero_v[pl.ds(i * 16, 16)] = jnp.zeros((16,), jnp.float32)
      return carry
    lax.fori_loop(0, zn if wide else zn // 16, zfill, 0)
    def zcopy(i, carry):
      pltpu.sync_copy(zero_v, acc_sh.at[pl.ds(zbase + i * zn, zn)])
      return carry
    lax.fori_loop(0, per_sub // zn, zcopy, 0)
    plsc.subcore_barrier()

    def scale(buf, j):
      def grp(g, carry):
        w16 = w_v[j, pl.ds(g * 16, 16)]
        if wide:
          for i in range(16):
            e = g * 16 + i
            buf[e, :] = buf[e, :] * w16[i]
        else:
          buf[pl.ds(g * 16, 16)] = buf[pl.ds(g * 16, 16)] * w16
        return carry
      lax.fori_loop(0, SROW // 16, grp, 0)

    def superstep(t, carry):
      rbase = (wid * n_super + t) * NSTR
      if gather:
        pltpu.sync_copy(src_hbm.at[pl.ds(rbase, NSTR)], src_v)
      pltpu.sync_copy(dst_hbm.at[pl.ds(rbase, NSTR)], dst_v)
      pltpu.sync_copy(w_hbm.at[pl.ds(rbase, NSTR)], w_v)

      if gather:
        gh = {}
        sh = {}
        gh[0] = pltpu.async_copy(h_hbm.at[src_v.at[0]], bufs[0], gsem)
        for j in range(NSTR):
          if j + 1 < NSTR:
            if j >= 1:
              sh[j - 1].wait()
            gh[j + 1] = pltpu.async_copy(
                h_hbm.at[src_v.at[j + 1]], bufs[(j + 1) % 2], gsem)
          gh[j].wait()
          scale(bufs[j % 2], j)
          sh[j] = pltpu.async_copy(
              bufs[j % 2], acc_sh.at[dst_v.at[j]], ssem, add=True)
        sh[NSTR - 2].wait()
        sh[NSTR - 1].wait()
      else:
        sh = {}
        for j in range(NSTR):
          if j >= 2:
            sh[j - 2].wait()
          sh[j] = pltpu.async_copy(
              w_v.at[j], acc_sh.at[dst_v.at[j]], ssem, add=True)
        sh[NSTR - 2].wait()
        sh[NSTR - 1].wait()
      return carry
    lax.fori_loop(0, n_super, superstep, 0)
    plsc.subcore_barrier()

    pltpu.sync_copy(acc_sh.at[pl.ds(zbase, per_sub)],
                    out_hbm.at[c].at[pl.ds(zbase, per_sub)])

  if gather:
    return kfn(src2d, dst2d, w2d, h)
  return kfn(dst2d, w2d)


def _tc_prep(deg_agg, x0, w1):
  n = x0.shape[0]
  blk = 1568
  grid = (n // blk,)

  def body(dega_ref, x0_ref, w1_ref, dinv_ref, h1p_ref):
    deg = dega_ref[0, :, :] + dega_ref[1, :, :] + 1.0
    dinv = lax.rsqrt(deg)
    dinv_ref[:, :] = dinv
    h1p_ref[:, :] = jnp.dot(x0_ref[:, :], w1_ref[:, :],
                            preferred_element_type=jnp.float32) * dinv

  return pl.pallas_call(
      body,
      grid=grid,
      in_specs=[
          pl.BlockSpec((NC, blk, 1), lambda i: (0, i, 0)),
          pl.BlockSpec((blk, 4), lambda i: (i, 0)),
          pl.BlockSpec((4, HID), lambda i: (0, 0)),
      ],
      out_specs=[
          pl.BlockSpec((blk, 1), lambda i: (i, 0)),
          pl.BlockSpec((blk, HID), lambda i: (i, 0)),
      ],
      out_shape=[
          jax.ShapeDtypeStruct((n, 1), jnp.float32),
          jax.ShapeDtypeStruct((n, HID), jnp.float32),
      ],
  )(deg_agg, x0, w1)


def _tc_layer(agg, hp, dinv, b, wnext):
  n = hp.shape[0]
  blk = 1568
  grid = (n // blk,)
  fo = wnext.shape[1]

  def body(agg_ref, hp_ref, dinv_ref, b_ref, wn_ref, out_ref):
    a = agg_ref[0, :, :] + agg_ref[1, :, :] + hp_ref[:, :]
    x = jnp.maximum(dinv_ref[:, :] * a + b_ref[:, :], 0.0)
    out_ref[:, :] = jnp.dot(x, wn_ref[:, :],
                            preferred_element_type=jnp.float32) * dinv_ref[:, :]

  return pl.pallas_call(
      body,
      grid=grid,
      in_specs=[
          pl.BlockSpec((NC, blk, HID), lambda i: (0, i, 0)),
          pl.BlockSpec((blk, HID), lambda i: (i, 0)),
          pl.BlockSpec((blk, 1), lambda i: (i, 0)),
          pl.BlockSpec((1, HID), lambda i: (0, 0)),
          pl.BlockSpec((HID, fo), lambda i: (0, 0)),
      ],
      out_specs=pl.BlockSpec((blk, fo), lambda i: (i, 0)),
      out_shape=jax.ShapeDtypeStruct((n, fo), jnp.float32),
  )(agg, hp, dinv, b, wnext)


def _tc_final(agg, hp, dinv, b3, n_real):
  n = hp.shape[0]
  blk = 1568
  grid = (n // blk,)

  def body(agg_ref, hp_ref, dinv_ref, b3_ref, out_ref):
    a = agg_ref[0, :, :] + agg_ref[1, :, :] + hp_ref[:, :]
    x = jnp.maximum(dinv_ref[:, :] * a + b3_ref[:, :], 0.0)
    rows = (lax.broadcasted_iota(jnp.int32, (blk, 1), 0)
            + pl.program_id(0) * blk)
    x = jnp.where(rows < n_real, x, 0.0)
    part = jnp.sum(x) * (1.0 / n_real)

    @pl.when(pl.program_id(0) == 0)
    def _():
      out_ref[:, :] = jnp.zeros((1, 1), jnp.float32)

    out_ref[:, :] = out_ref[:, :] + part

  return pl.pallas_call(
      body,
      grid=grid,
      in_specs=[
          pl.BlockSpec((NC, blk, 1), lambda i: (0, i, 0)),
          pl.BlockSpec((blk, 1), lambda i: (i, 0)),
          pl.BlockSpec((blk, 1), lambda i: (i, 0)),
          pl.BlockSpec((1, 1), lambda i: (0, 0)),
      ],
      out_specs=pl.BlockSpec((1, 1), lambda i: (0, 0)),
      out_shape=jax.ShapeDtypeStruct((1, 1), jnp.float32),
  )(agg, hp, dinv, b3)


def kernel(edges, weights, vertex_features, W1, b1, W2, b2, W3, b3):
  src = edges[0]
  dst = edges[1]
  n = vertex_features.shape[0]
  e = src.shape[0]

  n_super = -(-e // (NW * SUPER))
  e_pad = NW * n_super * SUPER
  pad = e_pad - e
  src_p = jnp.concatenate([src, jnp.zeros((pad,), jnp.int32)])
  dst_p = jnp.concatenate([dst, jnp.zeros((pad,), jnp.int32)])
  w_p = jnp.concatenate([weights, jnp.zeros((pad,), jnp.float32)])
  src2d = src_p.reshape(e_pad // SROW, SROW)
  dst2d = dst_p.reshape(e_pad // SROW, SROW)
  w2d = w_p.reshape(e_pad // SROW, SROW)

  x0_p = jnp.pad(vertex_features, ((0, N_PAD - n), (0, 0)))

  deg_agg = _sc_pass("deg", None, dst2d, w2d, None, n_super)

  dinv, h1p = _tc_prep(deg_agg.reshape(NC, N_PAD, 1), x0_p, W1)

  agg1 = _sc_pass("wide", src2d, dst2d, w2d, h1p, n_super)
  h2p = _tc_layer(agg1, h1p, dinv, b1.reshape(1, HID), W2)

  agg2 = _sc_pass("wide", src2d, dst2d, w2d, h2p, n_super)
  h3p = _tc_layer(agg2, h2p, dinv, b2.reshape(1, HID), W3)

  agg3 = _sc_pass("elem", src2d, dst2d, w2d, h3p.reshape(N_PAD), n_super)
  q = _tc_final(agg3.reshape(NC, N_PAD, 1), h3p, dinv, b3.reshape(1, 1), n)
  return q


# speedup vs baseline: 97.1466x; 1.7549x over previous
import functools

import jax
import jax.numpy as jnp
from jax import lax
from jax.experimental import pallas as pl
from jax.experimental.pallas import tpu as pltpu
from jax.experimental.pallas import tpu_sc as plsc

HID = 16
NC = 2
NS = 16
NW = NC * NS

ROW = 128
SUB = 4
NSTR = 4
STEPS = SUB * NSTR
SUPER = ROW * STEPS
SROW = SUB * ROW

N_PAD = 50176
ZROWS = 392
ZROWS1 = 448


def _sc_pass(variant, src2d, dst2d, w2d, h, n_super):
  wide = variant == "wide"
  gather = variant != "deg"
  n = N_PAD
  per_sub = n // NS

  mesh = plsc.VectorSubcoreMesh(core_axis_name="c", subcore_axis_name="s")

  if wide:
    acc_t = pltpu.VMEM_SHARED((n, HID), jnp.float32)
    out_t = jax.ShapeDtypeStruct((NC, n, HID), jnp.float32)
    buf_t = pltpu.VMEM((SROW, HID), jnp.float32)
  else:
    acc_t = pltpu.VMEM_SHARED((n,), jnp.float32)
    out_t = jax.ShapeDtypeStruct((NC, n), jnp.float32)
    buf_t = pltpu.VMEM((SROW,), jnp.float32)

  scratch = [
      acc_t,
  ] + ([pltpu.VMEM_SHARED((n, HID), jnp.float32)] if wide else []) + [
      pltpu.VMEM((NSTR, SROW), jnp.int32),
      pltpu.VMEM((NSTR, SROW), jnp.float32),
      buf_t, buf_t,
      pltpu.SemaphoreType.DMA,
      pltpu.SemaphoreType.DMA,
  ]
  if gather:
    scratch.insert(1, pltpu.VMEM((NSTR, SROW), jnp.int32))

  @functools.partial(
      pl.kernel,
      mesh=mesh,
      out_type=out_t,
      compiler_params=pltpu.CompilerParams(use_tc_tiling_on_sc=False),
      scratch_types=scratch,
  )
  def kfn(*refs):
    h_sh = None
    if wide:
      (src_hbm, dst_hbm, w_hbm, h_hbm, out_hbm,
       acc_sh, src_v, h_sh, dst_v, w_v, buf_a, buf_b,
       gsem, ssem) = refs
    elif gather:
      (src_hbm, dst_hbm, w_hbm, h_hbm, out_hbm,
       acc_sh, src_v, dst_v, w_v, buf_a, buf_b, gsem, ssem) = refs
    else:
      (dst_hbm, w_hbm, out_hbm,
       acc_sh, dst_v, w_v, buf_a, buf_b, gsem, ssem) = refs
    c = lax.axis_index("c")
    s = lax.axis_index("s")
    wid = c * NS + s
    bufs = (buf_a, buf_b)

    zbase = s * per_sub
    if wide:
      pltpu.sync_copy(h_hbm.at[pl.ds(zbase, per_sub)],
                      h_sh.at[pl.ds(zbase, per_sub)])
    zn = ZROWS if wide else ZROWS1
    def zfill(i, carry):
      if wide:
        buf_a[i, :] = jnp.zeros((HID,), jnp.float32)
      else:
        buf_a[pl.ds(i * 16, 16)] = jnp.zeros((16,), jnp.float32)
      return carry
    lax.fori_loop(0, zn if wide else zn // 16, zfill, 0)
    def zcopy(i, carry):
      if wide:
        pltpu.sync_copy(buf_a.at[pl.ds(0, zn)],
                        acc_sh.at[pl.ds(zbase + i * zn, zn)])
      else:
        pltpu.sync_copy(buf_a.at[pl.ds(0, zn)],
                        acc_sh.at[pl.ds(zbase + i * zn, zn)])
      return carry
    lax.fori_loop(0, per_sub // zn, zcopy, 0)
    plsc.subcore_barrier()

    def scale(buf, j):
      def grp(g, carry):
        w16 = w_v[j, pl.ds(g * 16, 16)]
        if wide:
          for i in range(16):
            e = g * 16 + i
            buf[e, :] = buf[e, :] * w16[i]
        else:
          buf[pl.ds(g * 16, 16)] = buf[pl.ds(g * 16, 16)] * w16
        return carry
      lax.fori_loop(0, SROW // 16, grp, 0)

    def superstep(t, carry):
      rbase = (wid * n_super + t) * NSTR
      if gather:
        pltpu.sync_copy(src_hbm.at[pl.ds(rbase, NSTR)], src_v)
      pltpu.sync_copy(dst_hbm.at[pl.ds(rbase, NSTR)], dst_v)
      pltpu.sync_copy(w_hbm.at[pl.ds(rbase, NSTR)], w_v)

      if gather:
        gh = {}
        sh = {}
        h_src = h_sh if wide else h_hbm
        gh[0] = pltpu.async_copy(h_src.at[src_v.at[0]], bufs[0], gsem)
        for j in range(NSTR):
          if j + 1 < NSTR:
            if j >= 1:
              sh[j - 1].wait()
            gh[j + 1] = pltpu.async_copy(
                h_src.at[src_v.at[j + 1]], bufs[(j + 1) % 2], gsem)
          gh[j].wait()
          scale(bufs[j % 2], j)
          sh[j] = pltpu.async_copy(
              bufs[j % 2], acc_sh.at[dst_v.at[j]], ssem, add=True)
        sh[NSTR - 2].wait()
        sh[NSTR - 1].wait()
      else:
        sh = {}
        for j in range(NSTR):
          if j >= 2:
            sh[j - 2].wait()
          sh[j] = pltpu.async_copy(
              w_v.at[j], acc_sh.at[dst_v.at[j]], ssem, add=True)
        sh[NSTR - 2].wait()
        sh[NSTR - 1].wait()
      return carry
    lax.fori_loop(0, n_super, superstep, 0)
    plsc.subcore_barrier()

    pltpu.sync_copy(acc_sh.at[pl.ds(zbase, per_sub)],
                    out_hbm.at[c].at[pl.ds(zbase, per_sub)])

  if gather:
    return kfn(src2d, dst2d, w2d, h)
  return kfn(dst2d, w2d)


def _tc_prep(deg_agg, x0, w1):
  n = x0.shape[0]
  blk = 1568
  grid = (n // blk,)

  def body(dega_ref, x0_ref, w1_ref, dinv_ref, h1p_ref):
    deg = dega_ref[0, :, :] + dega_ref[1, :, :] + 1.0
    dinv = lax.rsqrt(deg)
    dinv_ref[:, :] = dinv
    h1p_ref[:, :] = jnp.dot(x0_ref[:, :], w1_ref[:, :],
                            preferred_element_type=jnp.float32) * dinv

  return pl.pallas_call(
      body,
      grid=grid,
      in_specs=[
          pl.BlockSpec((NC, blk, 1), lambda i: (0, i, 0)),
          pl.BlockSpec((blk, 4), lambda i: (i, 0)),
          pl.BlockSpec((4, HID), lambda i: (0, 0)),
      ],
      out_specs=[
          pl.BlockSpec((blk, 1), lambda i: (i, 0)),
          pl.BlockSpec((blk, HID), lambda i: (i, 0)),
      ],
      out_shape=[
          jax.ShapeDtypeStruct((n, 1), jnp.float32),
          jax.ShapeDtypeStruct((n, HID), jnp.float32),
      ],
  )(deg_agg, x0, w1)


def _tc_layer(agg, hp, dinv, b, wnext):
  n = hp.shape[0]
  blk = 1568
  grid = (n // blk,)
  fo = wnext.shape[1]

  def body(agg_ref, hp_ref, dinv_ref, b_ref, wn_ref, out_ref):
    a = agg_ref[0, :, :] + agg_ref[1, :, :] + hp_ref[:, :]
    x = jnp.maximum(dinv_ref[:, :] * a + b_ref[:, :], 0.0)
    out_ref[:, :] = jnp.dot(x, wn_ref[:, :],
                            preferred_element_type=jnp.float32) * dinv_ref[:, :]

  return pl.pallas_call(
      body,
      grid=grid,
      in_specs=[
          pl.BlockSpec((NC, blk, HID), lambda i: (0, i, 0)),
          pl.BlockSpec((blk, HID), lambda i: (i, 0)),
          pl.BlockSpec((blk, 1), lambda i: (i, 0)),
          pl.BlockSpec((1, HID), lambda i: (0, 0)),
          pl.BlockSpec((HID, fo), lambda i: (0, 0)),
      ],
      out_specs=pl.BlockSpec((blk, fo), lambda i: (i, 0)),
      out_shape=jax.ShapeDtypeStruct((n, fo), jnp.float32),
  )(agg, hp, dinv, b, wnext)


def _tc_final(agg, hp, dinv, b3, n_real):
  n = hp.shape[0]
  blk = 1568
  grid = (n // blk,)

  def body(agg_ref, hp_ref, dinv_ref, b3_ref, out_ref):
    a = agg_ref[0, :, :] + agg_ref[1, :, :] + hp_ref[:, :]
    x = jnp.maximum(dinv_ref[:, :] * a + b3_ref[:, :], 0.0)
    rows = (lax.broadcasted_iota(jnp.int32, (blk, 1), 0)
            + pl.program_id(0) * blk)
    x = jnp.where(rows < n_real, x, 0.0)
    part = jnp.sum(x) * (1.0 / n_real)

    @pl.when(pl.program_id(0) == 0)
    def _():
      out_ref[:, :] = jnp.zeros((1, 1), jnp.float32)

    out_ref[:, :] = out_ref[:, :] + part

  return pl.pallas_call(
      body,
      grid=grid,
      in_specs=[
          pl.BlockSpec((NC, blk, 1), lambda i: (0, i, 0)),
          pl.BlockSpec((blk, 1), lambda i: (i, 0)),
          pl.BlockSpec((blk, 1), lambda i: (i, 0)),
          pl.BlockSpec((1, 1), lambda i: (0, 0)),
      ],
      out_specs=pl.BlockSpec((1, 1), lambda i: (0, 0)),
      out_shape=jax.ShapeDtypeStruct((1, 1), jnp.float32),
  )(agg, hp, dinv, b3)


def kernel(edges, weights, vertex_features, W1, b1, W2, b2, W3, b3):
  src = edges[0]
  dst = edges[1]
  n = vertex_features.shape[0]
  e = src.shape[0]

  n_super = -(-e // (NW * SUPER))
  e_pad = NW * n_super * SUPER
  pad = e_pad - e
  src_p = jnp.concatenate([src, jnp.zeros((pad,), jnp.int32)])
  dst_p = jnp.concatenate([dst, jnp.zeros((pad,), jnp.int32)])
  w_p = jnp.concatenate([weights, jnp.zeros((pad,), jnp.float32)])
  src2d = src_p.reshape(e_pad // SROW, SROW)
  dst2d = dst_p.reshape(e_pad // SROW, SROW)
  w2d = w_p.reshape(e_pad // SROW, SROW)

  x0_p = jnp.pad(vertex_features, ((0, N_PAD - n), (0, 0)))

  deg_agg = _sc_pass("deg", None, dst2d, w2d, None, n_super)

  dinv, h1p = _tc_prep(deg_agg.reshape(NC, N_PAD, 1), x0_p, W1)

  agg1 = _sc_pass("wide", src2d, dst2d, w2d, h1p, n_super)
  h2p = _tc_layer(agg1, h1p, dinv, b1.reshape(1, HID), W2)

  agg2 = _sc_pass("wide", src2d, dst2d, w2d, h2p, n_super)
  h3p = _tc_layer(agg2, h2p, dinv, b2.reshape(1, HID), W3)

  agg3 = _sc_pass("elem", src2d, dst2d, w2d, h3p.reshape(N_PAD), n_super)
  q = _tc_final(agg3.reshape(NC, N_PAD, 1), h3p, dinv, b3.reshape(1, 1), n)
  return q


# speedup vs baseline: 108.4860x; 1.1167x over previous
import functools

import jax
import jax.numpy as jnp
from jax import lax
from jax.experimental import pallas as pl
from jax.experimental.pallas import tpu as pltpu
from jax.experimental.pallas import tpu_sc as plsc

HID = 16
NC = 2
NS = 16
NW = NC * NS

ROW = 128
SUB = 4
NSTR = 4
STEPS = SUB * NSTR
SUPER = ROW * STEPS
SROW = SUB * ROW

N_PAD = 50176
ZROWS = 392
ZROWS1 = 448


def _sc_pass(variant, src2d, dst2d, w2d, h, n_super):
  wide = variant == "wide"
  gather = variant != "deg"
  n = N_PAD
  per_sub = n // NS

  mesh = plsc.VectorSubcoreMesh(core_axis_name="c", subcore_axis_name="s")

  if wide:
    acc_t = pltpu.VMEM_SHARED((n, HID), jnp.float32)
    out_t = jax.ShapeDtypeStruct((NC, n, HID), jnp.float32)
    buf_t = pltpu.VMEM((SROW, HID), jnp.float32)
  else:
    acc_t = pltpu.VMEM_SHARED((n,), jnp.float32)
    out_t = jax.ShapeDtypeStruct((NC, n), jnp.float32)
    buf_t = pltpu.VMEM((SROW,), jnp.float32)

  h_sh_t = (pltpu.VMEM_SHARED((n, HID), jnp.float32) if wide
            else pltpu.VMEM_SHARED((n,), jnp.float32))
  scratch = [
      acc_t,
  ] + ([h_sh_t] if gather else []) + [
      pltpu.VMEM((NSTR, SROW), jnp.int32),
      pltpu.VMEM((NSTR, SROW), jnp.float32),
      buf_t, buf_t,
      pltpu.SemaphoreType.DMA,
      pltpu.SemaphoreType.DMA,
  ]
  if gather:
    scratch.insert(1, pltpu.VMEM((NSTR, SROW), jnp.int32))

  @functools.partial(
      pl.kernel,
      mesh=mesh,
      out_type=out_t,
      compiler_params=pltpu.CompilerParams(use_tc_tiling_on_sc=False),
      scratch_types=scratch,
  )
  def kfn(*refs):
    h_sh = None
    if wide:
      (src_hbm, dst_hbm, w_hbm, h_hbm, out_hbm,
       acc_sh, src_v, h_sh, dst_v, w_v, buf_a, buf_b,
       gsem, ssem) = refs
    elif gather:
      (src_hbm, dst_hbm, w_hbm, h_hbm, out_hbm,
       acc_sh, src_v, h_sh, dst_v, w_v, buf_a, buf_b, gsem, ssem) = refs
    else:
      (dst_hbm, w_hbm, out_hbm,
       acc_sh, dst_v, w_v, buf_a, buf_b, gsem, ssem) = refs
    c = lax.axis_index("c")
    s = lax.axis_index("s")
    wid = c * NS + s
    bufs = (buf_a, buf_b)

    zbase = s * per_sub
    if gather:
      pltpu.sync_copy(h_hbm.at[pl.ds(zbase, per_sub)],
                      h_sh.at[pl.ds(zbase, per_sub)])
    zn = ZROWS if wide else ZROWS1
    def zfill(i, carry):
      if wide:
        buf_a[i, :] = jnp.zeros((HID,), jnp.float32)
      else:
        buf_a[pl.ds(i * 16, 16)] = jnp.zeros((16,), jnp.float32)
      return carry
    lax.fori_loop(0, zn if wide else zn // 16, zfill, 0)
    def zcopy(i, carry):
      if wide:
        pltpu.sync_copy(buf_a.at[pl.ds(0, zn)],
                        acc_sh.at[pl.ds(zbase + i * zn, zn)])
      else:
        pltpu.sync_copy(buf_a.at[pl.ds(0, zn)],
                        acc_sh.at[pl.ds(zbase + i * zn, zn)])
      return carry
    lax.fori_loop(0, per_sub // zn, zcopy, 0)
    plsc.subcore_barrier()

    def scale(buf, j):
      def grp(g, carry):
        w16 = w_v[j, pl.ds(g * 16, 16)]
        if wide:
          for i in range(16):
            e = g * 16 + i
            buf[e, :] = buf[e, :] * w16[i]
        else:
          buf[pl.ds(g * 16, 16)] = buf[pl.ds(g * 16, 16)] * w16
        return carry
      lax.fori_loop(0, SROW // 16, grp, 0)

    def superstep(t, carry):
      rbase = (wid * n_super + t) * NSTR
      if gather:
        pltpu.sync_copy(src_hbm.at[pl.ds(rbase, NSTR)], src_v)
      pltpu.sync_copy(dst_hbm.at[pl.ds(rbase, NSTR)], dst_v)
      pltpu.sync_copy(w_hbm.at[pl.ds(rbase, NSTR)], w_v)

      if gather:
        gh = {}
        sh = {}
        h_src = h_sh
        gh[0] = pltpu.async_copy(h_src.at[src_v.at[0]], bufs[0], gsem)
        for j in range(NSTR):
          if j + 1 < NSTR:
            if j >= 1:
              sh[j - 1].wait()
            gh[j + 1] = pltpu.async_copy(
                h_src.at[src_v.at[j + 1]], bufs[(j + 1) % 2], gsem)
          gh[j].wait()
          scale(bufs[j % 2], j)
          sh[j] = pltpu.async_copy(
              bufs[j % 2], acc_sh.at[dst_v.at[j]], ssem, add=True)
        sh[NSTR - 2].wait()
        sh[NSTR - 1].wait()
      else:
        sh = {}
        for j in range(NSTR):
          if j >= 2:
            sh[j - 2].wait()
          sh[j] = pltpu.async_copy(
              w_v.at[j], acc_sh.at[dst_v.at[j]], ssem, add=True)
        sh[NSTR - 2].wait()
        sh[NSTR - 1].wait()
      return carry
    lax.fori_loop(0, n_super, superstep, 0)
    plsc.subcore_barrier()

    pltpu.sync_copy(acc_sh.at[pl.ds(zbase, per_sub)],
                    out_hbm.at[c].at[pl.ds(zbase, per_sub)])

  if gather:
    return kfn(src2d, dst2d, w2d, h)
  return kfn(dst2d, w2d)


def _tc_prep(deg_agg, x0, w1):
  n = x0.shape[0]
  blk = 1568
  grid = (n // blk,)

  def body(dega_ref, x0_ref, w1_ref, dinv_ref, h1p_ref):
    deg = dega_ref[0, :, :] + dega_ref[1, :, :] + 1.0
    dinv = lax.rsqrt(deg)
    dinv_ref[:, :] = dinv
    h1p_ref[:, :] = jnp.dot(x0_ref[:, :], w1_ref[:, :],
                            preferred_element_type=jnp.float32) * dinv

  return pl.pallas_call(
      body,
      grid=grid,
      in_specs=[
          pl.BlockSpec((NC, blk, 1), lambda i: (0, i, 0)),
          pl.BlockSpec((blk, 4), lambda i: (i, 0)),
          pl.BlockSpec((4, HID), lambda i: (0, 0)),
      ],
      out_specs=[
          pl.BlockSpec((blk, 1), lambda i: (i, 0)),
          pl.BlockSpec((blk, HID), lambda i: (i, 0)),
      ],
      out_shape=[
          jax.ShapeDtypeStruct((n, 1), jnp.float32),
          jax.ShapeDtypeStruct((n, HID), jnp.float32),
      ],
  )(deg_agg, x0, w1)


def _tc_layer(agg, hp, dinv, b, wnext):
  n = hp.shape[0]
  blk = 1568
  grid = (n // blk,)
  fo = wnext.shape[1]

  def body(agg_ref, hp_ref, dinv_ref, b_ref, wn_ref, out_ref):
    a = agg_ref[0, :, :] + agg_ref[1, :, :] + hp_ref[:, :]
    x = jnp.maximum(dinv_ref[:, :] * a + b_ref[:, :], 0.0)
    out_ref[:, :] = jnp.dot(x, wn_ref[:, :],
                            preferred_element_type=jnp.float32) * dinv_ref[:, :]

  return pl.pallas_call(
      body,
      grid=grid,
      in_specs=[
          pl.BlockSpec((NC, blk, HID), lambda i: (0, i, 0)),
          pl.BlockSpec((blk, HID), lambda i: (i, 0)),
          pl.BlockSpec((blk, 1), lambda i: (i, 0)),
          pl.BlockSpec((1, HID), lambda i: (0, 0)),
          pl.BlockSpec((HID, fo), lambda i: (0, 0)),
      ],
      out_specs=pl.BlockSpec((blk, fo), lambda i: (i, 0)),
      out_shape=jax.ShapeDtypeStruct((n, fo), jnp.float32),
  )(agg, hp, dinv, b, wnext)


def _tc_final(agg, hp, dinv, b3, n_real):
  n = hp.shape[0]
  blk = 1568
  grid = (n // blk,)

  def body(agg_ref, hp_ref, dinv_ref, b3_ref, out_ref):
    a = agg_ref[0, :, :] + agg_ref[1, :, :] + hp_ref[:, :]
    x = jnp.maximum(dinv_ref[:, :] * a + b3_ref[:, :], 0.0)
    rows = (lax.broadcasted_iota(jnp.int32, (blk, 1), 0)
            + pl.program_id(0) * blk)
    x = jnp.where(rows < n_real, x, 0.0)
    part = jnp.sum(x) * (1.0 / n_real)

    @pl.when(pl.program_id(0) == 0)
    def _():
      out_ref[:, :] = jnp.zeros((1, 1), jnp.float32)

    out_ref[:, :] = out_ref[:, :] + part

  return pl.pallas_call(
      body,
      grid=grid,
      in_specs=[
          pl.BlockSpec((NC, blk, 1), lambda i: (0, i, 0)),
          pl.BlockSpec((blk, 1), lambda i: (i, 0)),
          pl.BlockSpec((blk, 1), lambda i: (i, 0)),
          pl.BlockSpec((1, 1), lambda i: (0, 0)),
      ],
      out_specs=pl.BlockSpec((1, 1), lambda i: (0, 0)),
      out_shape=jax.ShapeDtypeStruct((1, 1), jnp.float32),
  )(agg, hp, dinv, b3)


def kernel(edges, weights, vertex_features, W1, b1, W2, b2, W3, b3):
  src = edges[0]
  dst = edges[1]
  n = vertex_features.shape[0]
  e = src.shape[0]

  n_super = -(-e // (NW * SUPER))
  e_pad = NW * n_super * SUPER
  pad = e_pad - e
  src_p = jnp.concatenate([src, jnp.zeros((pad,), jnp.int32)])
  dst_p = jnp.concatenate([dst, jnp.zeros((pad,), jnp.int32)])
  w_p = jnp.concatenate([weights, jnp.zeros((pad,), jnp.float32)])
  src2d = src_p.reshape(e_pad // SROW, SROW)
  dst2d = dst_p.reshape(e_pad // SROW, SROW)
  w2d = w_p.reshape(e_pad // SROW, SROW)

  x0_p = jnp.pad(vertex_features, ((0, N_PAD - n), (0, 0)))

  deg_agg = _sc_pass("deg", None, dst2d, w2d, None, n_super)

  dinv, h1p = _tc_prep(deg_agg.reshape(NC, N_PAD, 1), x0_p, W1)

  agg1 = _sc_pass("wide", src2d, dst2d, w2d, h1p, n_super)
  h2p = _tc_layer(agg1, h1p, dinv, b1.reshape(1, HID), W2)

  agg2 = _sc_pass("wide", src2d, dst2d, w2d, h2p, n_super)
  h3p = _tc_layer(agg2, h2p, dinv, b2.reshape(1, HID), W3)

  agg3 = _sc_pass("elem", src2d, dst2d, w2d, h3p.reshape(N_PAD), n_super)
  q = _tc_final(agg3.reshape(NC, N_PAD, 1), h3p, dinv, b3.reshape(1, 1), n)
  return q


# speedup vs baseline: 109.5666x; 1.0100x over previous
import functools

import jax
import jax.numpy as jnp
from jax import lax
from jax.experimental import pallas as pl
from jax.experimental.pallas import tpu as pltpu
from jax.experimental.pallas import tpu_sc as plsc

HID = 16
NC = 2
NS = 16
NW = NC * NS

ROW = 128
SUB = 4
NSTR = 4
STEPS = SUB * NSTR
SUPER = ROW * STEPS
SROW = SUB * ROW

N_PAD = 50176
ZROWS = 392
ZROWS1 = 448


def _sc_pass(variant, src2d, dst2d, w2d, h, n_super):
  wide = variant == "wide"
  gather = variant != "deg"
  n = N_PAD
  per_sub = n // NS

  mesh = plsc.VectorSubcoreMesh(core_axis_name="c", subcore_axis_name="s")

  if wide:
    acc_t = pltpu.VMEM_SHARED((n, HID), jnp.float32)
    out_t = jax.ShapeDtypeStruct((NC, n, HID), jnp.float32)
    buf_t = pltpu.VMEM((SROW, HID), jnp.float32)
  else:
    acc_t = pltpu.VMEM_SHARED((n,), jnp.float32)
    out_t = jax.ShapeDtypeStruct((NC, n), jnp.float32)
    buf_t = pltpu.VMEM((SROW,), jnp.float32)

  h_sh_t = (pltpu.VMEM_SHARED((n, HID), jnp.float32) if wide
            else pltpu.VMEM_SHARED((n,), jnp.float32))
  scratch = [
      acc_t,
  ] + ([h_sh_t] if gather else []) + [
      pltpu.VMEM((NSTR, SROW), jnp.int32),
      pltpu.VMEM((NSTR, SROW), jnp.float32),
      buf_t, buf_t,
      pltpu.SemaphoreType.DMA,
      pltpu.SemaphoreType.DMA,
  ]
  if gather:
    scratch.insert(1, pltpu.VMEM((NSTR, SROW), jnp.int32))

  @functools.partial(
      pl.kernel,
      mesh=mesh,
      out_type=out_t,
      compiler_params=pltpu.CompilerParams(use_tc_tiling_on_sc=False),
      scratch_types=scratch,
  )
  def kfn(*refs):
    h_sh = None
    if wide:
      (src_hbm, dst_hbm, w_hbm, h_hbm, out_hbm,
       acc_sh, src_v, h_sh, dst_v, w_v, buf_a, buf_b,
       gsem, ssem) = refs
    elif gather:
      (src_hbm, dst_hbm, w_hbm, h_hbm, out_hbm,
       acc_sh, src_v, h_sh, dst_v, w_v, buf_a, buf_b, gsem, ssem) = refs
    else:
      (dst_hbm, w_hbm, out_hbm,
       acc_sh, dst_v, w_v, buf_a, buf_b, gsem, ssem) = refs
    c = lax.axis_index("c")
    s = lax.axis_index("s")
    wid = c * NS + s
    bufs = (buf_a, buf_b)

    zbase = s * per_sub
    if gather:
      pltpu.sync_copy(h_hbm.at[pl.ds(zbase, per_sub)],
                      h_sh.at[pl.ds(zbase, per_sub)])
    zn = ZROWS if wide else ZROWS1
    def zfill(i, carry):
      if wide:
        buf_a[i, :] = jnp.zeros((HID,), jnp.float32)
      else:
        buf_a[pl.ds(i * 16, 16)] = jnp.zeros((16,), jnp.float32)
      return carry
    lax.fori_loop(0, zn if wide else zn // 16, zfill, 0)
    def zcopy(i, carry):
      if wide:
        pltpu.sync_copy(buf_a.at[pl.ds(0, zn)],
                        acc_sh.at[pl.ds(zbase + i * zn, zn)])
      else:
        pltpu.sync_copy(buf_a.at[pl.ds(0, zn)],
                        acc_sh.at[pl.ds(zbase + i * zn, zn)])
      return carry
    lax.fori_loop(0, per_sub // zn, zcopy, 0)
    plsc.subcore_barrier()

    def scale(buf, j):
      @plsc.parallel_loop(0, SROW // 16, unroll=2)
      def grp(g):
        w16 = w_v[j, pl.ds(g * 16, 16)]
        if wide:
          for i in range(16):
            e = g * 16 + i
            buf[e, :] = buf[e, :] * w16[i]
        else:
          buf[pl.ds(g * 16, 16)] = buf[pl.ds(g * 16, 16)] * w16

    def superstep(t, carry):
      rbase = (wid * n_super + t) * NSTR
      if gather:
        pltpu.sync_copy(src_hbm.at[pl.ds(rbase, NSTR)], src_v)
      pltpu.sync_copy(dst_hbm.at[pl.ds(rbase, NSTR)], dst_v)
      pltpu.sync_copy(w_hbm.at[pl.ds(rbase, NSTR)], w_v)

      if gather:
        gh = {}
        sh = {}
        h_src = h_sh
        gh[0] = pltpu.async_copy(h_src.at[src_v.at[0]], bufs[0], gsem)
        for j in range(NSTR):
          if j + 1 < NSTR:
            if j >= 1:
              sh[j - 1].wait()
            gh[j + 1] = pltpu.async_copy(
                h_src.at[src_v.at[j + 1]], bufs[(j + 1) % 2], gsem)
          gh[j].wait()
          scale(bufs[j % 2], j)
          sh[j] = pltpu.async_copy(
              bufs[j % 2], acc_sh.at[dst_v.at[j]], ssem, add=True)
        sh[NSTR - 2].wait()
        sh[NSTR - 1].wait()
      else:
        sh = {}
        for j in range(NSTR):
          if j >= 2:
            sh[j - 2].wait()
          sh[j] = pltpu.async_copy(
              w_v.at[j], acc_sh.at[dst_v.at[j]], ssem, add=True)
        sh[NSTR - 2].wait()
        sh[NSTR - 1].wait()
      return carry
    lax.fori_loop(0, n_super, superstep, 0)
    plsc.subcore_barrier()

    pltpu.sync_copy(acc_sh.at[pl.ds(zbase, per_sub)],
                    out_hbm.at[c].at[pl.ds(zbase, per_sub)])

  if gather:
    return kfn(src2d, dst2d, w2d, h)
  return kfn(dst2d, w2d)


def _tc_prep(deg_agg, x0, w1):
  n = x0.shape[0]
  blk = 1568
  grid = (n // blk,)

  def body(dega_ref, x0_ref, w1_ref, dinv_ref, h1p_ref):
    deg = dega_ref[0, :, :] + dega_ref[1, :, :] + 1.0
    dinv = lax.rsqrt(deg)
    dinv_ref[:, :] = dinv
    h1p_ref[:, :] = jnp.dot(x0_ref[:, :], w1_ref[:, :],
                            preferred_element_type=jnp.float32) * dinv

  return pl.pallas_call(
      body,
      grid=grid,
      in_specs=[
          pl.BlockSpec((NC, blk, 1), lambda i: (0, i, 0)),
          pl.BlockSpec((blk, 4), lambda i: (i, 0)),
          pl.BlockSpec((4, HID), lambda i: (0, 0)),
      ],
      out_specs=[
          pl.BlockSpec((blk, 1), lambda i: (i, 0)),
          pl.BlockSpec((blk, HID), lambda i: (i, 0)),
      ],
      out_shape=[
          jax.ShapeDtypeStruct((n, 1), jnp.float32),
          jax.ShapeDtypeStruct((n, HID), jnp.float32),
      ],
  )(deg_agg, x0, w1)


def _tc_layer(agg, hp, dinv, b, wnext):
  n = hp.shape[0]
  blk = 1568
  grid = (n // blk,)
  fo = wnext.shape[1]

  def body(agg_ref, hp_ref, dinv_ref, b_ref, wn_ref, out_ref):
    a = agg_ref[0, :, :] + agg_ref[1, :, :] + hp_ref[:, :]
    x = jnp.maximum(dinv_ref[:, :] * a + b_ref[:, :], 0.0)
    out_ref[:, :] = jnp.dot(x, wn_ref[:, :],
                            preferred_element_type=jnp.float32) * dinv_ref[:, :]

  return pl.pallas_call(
      body,
      grid=grid,
      in_specs=[
          pl.BlockSpec((NC, blk, HID), lambda i: (0, i, 0)),
          pl.BlockSpec((blk, HID), lambda i: (i, 0)),
          pl.BlockSpec((blk, 1), lambda i: (i, 0)),
          pl.BlockSpec((1, HID), lambda i: (0, 0)),
          pl.BlockSpec((HID, fo), lambda i: (0, 0)),
      ],
      out_specs=pl.BlockSpec((blk, fo), lambda i: (i, 0)),
      out_shape=jax.ShapeDtypeStruct((n, fo), jnp.float32),
  )(agg, hp, dinv, b, wnext)


def _tc_final(agg, hp, dinv, b3, n_real):
  n = hp.shape[0]
  blk = 1568
  grid = (n // blk,)

  def body(agg_ref, hp_ref, dinv_ref, b3_ref, out_ref):
    a = agg_ref[0, :, :] + agg_ref[1, :, :] + hp_ref[:, :]
    x = jnp.maximum(dinv_ref[:, :] * a + b3_ref[:, :], 0.0)
    rows = (lax.broadcasted_iota(jnp.int32, (blk, 1), 0)
            + pl.program_id(0) * blk)
    x = jnp.where(rows < n_real, x, 0.0)
    part = jnp.sum(x) * (1.0 / n_real)

    @pl.when(pl.program_id(0) == 0)
    def _():
      out_ref[:, :] = jnp.zeros((1, 1), jnp.float32)

    out_ref[:, :] = out_ref[:, :] + part

  return pl.pallas_call(
      body,
      grid=grid,
      in_specs=[
          pl.BlockSpec((NC, blk, 1), lambda i: (0, i, 0)),
          pl.BlockSpec((blk, 1), lambda i: (i, 0)),
          pl.BlockSpec((blk, 1), lambda i: (i, 0)),
          pl.BlockSpec((1, 1), lambda i: (0, 0)),
      ],
      out_specs=pl.BlockSpec((1, 1), lambda i: (0, 0)),
      out_shape=jax.ShapeDtypeStruct((1, 1), jnp.float32),
  )(agg, hp, dinv, b3)


def kernel(edges, weights, vertex_features, W1, b1, W2, b2, W3, b3):
  src = edges[0]
  dst = edges[1]
  n = vertex_features.shape[0]
  e = src.shape[0]

  n_super = -(-e // (NW * SUPER))
  e_pad = NW * n_super * SUPER
  pad = e_pad - e
  src_p = jnp.concatenate([src, jnp.zeros((pad,), jnp.int32)])
  dst_p = jnp.concatenate([dst, jnp.zeros((pad,), jnp.int32)])
  w_p = jnp.concatenate([weights, jnp.zeros((pad,), jnp.float32)])
  src2d = src_p.reshape(e_pad // SROW, SROW)
  dst2d = dst_p.reshape(e_pad // SROW, SROW)
  w2d = w_p.reshape(e_pad // SROW, SROW)

  x0_p = jnp.pad(vertex_features, ((0, N_PAD - n), (0, 0)))

  deg_agg = _sc_pass("deg", None, dst2d, w2d, None, n_super)

  dinv, h1p = _tc_prep(deg_agg.reshape(NC, N_PAD, 1), x0_p, W1)

  agg1 = _sc_pass("wide", src2d, dst2d, w2d, h1p, n_super)
  h2p = _tc_layer(agg1, h1p, dinv, b1.reshape(1, HID), W2)

  agg2 = _sc_pass("wide", src2d, dst2d, w2d, h2p, n_super)
  h3p = _tc_layer(agg2, h2p, dinv, b2.reshape(1, HID), W3)

  agg3 = _sc_pass("elem", src2d, dst2d, w2d, h3p.reshape(N_PAD), n_super)
  q = _tc_final(agg3.reshape(NC, N_PAD, 1), h3p, dinv, b3.reshape(1, 1), n)
  return q
